# baseline (speedup 1.0000x reference)
import numpy as np

import jax
import jax.numpy as jnp
from jax import lax
from jax.experimental import pallas as pl

IMG_H = 800.0
IMG_W = 800.0
FEAT_H = 50
FEAT_W = 50
C = 256
NA = 9
PRENMS_TOPK = 6000
POSTNMS_TOPK = 300
NMS_IOU = 0.7
BOX_MIN_SIZE = 16.0

_PW = 52
_FLAT = _PW * _PW
_XBUF_W = 2944
_XOFF = 64
_YW = 2816

_CLIP = np.float32(np.log(1000.0 / 16.0))

_SCALES = (128.0, 256.0, 512.0)
_RATIOS = (0.5, 1.0, 2.0)
_BASE = []
for _s in _SCALES:
    for _r in _RATIOS:
        _h = _s * np.sqrt(_r)
        _w = _s / np.sqrt(_r)
        _BASE.append((np.float32(-_w / 2.0), np.float32(-_h / 2.0),
                      np.float32(_w / 2.0), np.float32(_h / 2.0)))

_NMS_B = 1024
_NMS_NB = 6
_NMS_PAD = _NMS_NB * _NMS_B


def _head_body(xim_ref, wim_ref, wrc_ref, brpn_ref, brc_ref, out_ref):
    acc = lax.dot_general(wim_ref[:], xim_ref[:], (((1,), (0,)), ((), ())),
                          preferred_element_type=jnp.float32)
    h = jnp.maximum(acc + brpn_ref[:], 0.0)
    rc = lax.dot_general(wrc_ref[:], h, (((1,), (0,)), ((), ())),
                         preferred_element_type=jnp.float32) + brc_ref[:]

    jj = lax.broadcasted_iota(jnp.int32, (1, _YW), 1)
    gx = (16 * (jj % _PW - 1)).astype(jnp.float32)
    gy = (16 * (jj // _PW - 1)).astype(jnp.float32)

    for a in range(NA):
        bx1, by1, bx2, by2 = _BASE[a]
        x1a = gx + bx1
        y1a = gy + by1
        x2a = gx + bx2
        y2a = gy + by2
        wa = x2a - x1a
        ha = y2a - y1a
        cxa = x1a + 0.5 * wa
        cya = y1a + 0.5 * ha
        dx = rc[4 * a + 0:4 * a + 1, :]
        dy = rc[4 * a + 1:4 * a + 2, :]
        dw = jnp.minimum(rc[4 * a + 2:4 * a + 3, :], _CLIP)
        dh = jnp.minimum(rc[4 * a + 3:4 * a + 4, :], _CLIP)
        cx = cxa + dx * wa
        cy = cya + dy * ha
        w = wa * jnp.exp(dw)
        h2 = ha * jnp.exp(dh)
        x1 = jnp.minimum(jnp.maximum(cx - 0.5 * w, 0.0), IMG_W)
        y1 = jnp.minimum(jnp.maximum(cy - 0.5 * h2, 0.0), IMG_H)
        x2 = jnp.minimum(jnp.maximum(cx + 0.5 * w, 0.0), IMG_W)
        y2 = jnp.minimum(jnp.maximum(cy + 0.5 * h2, 0.0), IMG_H)
        logit = rc[4 * NA + a:4 * NA + a + 1, :]
        score = 1.0 / (1.0 + jnp.exp(-logit))
        out_ref[a:a + 1, :] = score
        out_ref[NA + a:NA + a + 1, :] = x1
        out_ref[2 * NA + a:2 * NA + a + 1, :] = y1
        out_ref[3 * NA + a:3 * NA + a + 1, :] = x2
        out_ref[4 * NA + a:4 * NA + a + 1, :] = y2


def _nms_body(x1c, y1c, x2c, y2c, x1r, y1r, x2r, y2r, keep_ref):
    keep_ref[:, :, :] = jnp.ones((_NMS_NB, _NMS_B, 1), jnp.float32)
    ii = lax.broadcasted_iota(jnp.int32, (_NMS_B, _NMS_B), 0)
    jj = lax.broadcasted_iota(jnp.int32, (_NMS_B, _NMS_B), 1)
    lt = (ii < jj).astype(jnp.float32)

    def matvec(m, k):
        return lax.dot_general(m, k, (((0,), (0,)), ((), ())),
                               preferred_element_type=jnp.float32)

    def outer(bi, _):
        ax1 = x1c[bi]
        ay1 = y1c[bi]
        ax2 = x2c[bi]
        ay2 = y2c[bi]
        area_a = jnp.maximum(ax2 - ax1, 0.0) * jnp.maximum(ay2 - ay1, 0.0)

        def iou_mask(bj):
            bx1 = x1r[bj]
            by1 = y1r[bj]
            bx2 = x2r[bj]
            by2 = y2r[bj]
            area_b = jnp.maximum(bx2 - bx1, 0.0) * jnp.maximum(by2 - by1, 0.0)
            iw = jnp.maximum(jnp.minimum(ax2, bx2) - jnp.maximum(ax1, bx1), 0.0)
            ih = jnp.maximum(jnp.minimum(ay2, by2) - jnp.maximum(ay1, by1), 0.0)
            inter = iw * ih
            iou = inter / (area_a + area_b - inter + 1e-9)
            return (iou > NMS_IOU).astype(jnp.float32)

        m_strict = iou_mask(bi) * lt
        k0 = keep_ref[bi]

        def fix_cond(c):
            kp, k = c
            return jnp.any(kp != k)

        def fix_body(c):
            _, k = c
            kn = jnp.where(matvec(m_strict, k) > 0.5, 0.0, k0)
            return (k, kn)

        first = jnp.where(matvec(m_strict, k0) > 0.5, 0.0, k0)
        _, k_fin = lax.while_loop(fix_cond, fix_body, (k0, first))
        keep_ref[bi] = k_fin

        def cross(bj, _c):
            supp = matvec(iou_mask(bj), k_fin)
            keep_ref[bj] = jnp.where(supp > 0.5, 0.0, keep_ref[bj])
            return 0

        lax.fori_loop(bi + 1, _NMS_NB, cross, 0)
        return 0

    lax.fori_loop(0, _NMS_NB, outer, 0)


def kernel(image, feat_map, target_bboxes, W_rpn, b_rpn, W_reg, b_reg,
           W_cls, b_cls):
    x = feat_map[0]
    xpad = jnp.pad(x, ((0, 0), (1, 1), (1, 1))).reshape(C, _FLAT)
    xbuf = jnp.pad(xpad, ((0, 0), (_XOFF, _XBUF_W - _FLAT - _XOFF)))
    xim = jnp.concatenate(
        [xbuf[:, _XOFF + (ky - 1) * _PW + (kx - 1):
                 _XOFF + (ky - 1) * _PW + (kx - 1) + _YW]
         for ky in range(3) for kx in range(3)], axis=0)
    wim = W_rpn.transpose(0, 2, 3, 1).reshape(C, 9 * C)
    wrc = jnp.concatenate([W_reg[:, :, 0, 0], W_cls[:, :, 0, 0]], axis=0)
    wrc = jnp.pad(wrc, ((0, 3), (0, 0)))
    brc = jnp.pad(jnp.concatenate([b_reg, b_cls]), (0, 3)).reshape(48, 1)
    brpn = b_rpn.reshape(C, 1)

    head = pl.pallas_call(
        _head_body,
        out_shape=jax.ShapeDtypeStruct((5 * NA, _YW), jnp.float32),
    )(xim, wim, wrc, brpn, brc)

    interior = head[:, :_FLAT].reshape(5 * NA, _PW, _PW)[:, 1:51, 1:51]

    def flat(q):
        return interior[q * NA:(q + 1) * NA].transpose(1, 2, 0).reshape(-1)

    scores = flat(0)
    boxes_all = jnp.stack([flat(1), flat(2), flat(3), flat(4)], axis=1)

    top_s, top_i = lax.top_k(scores, PRENMS_TOPK)
    boxes = boxes_all[top_i]
    ws = boxes[:, 2] - boxes[:, 0]
    hs = boxes[:, 3] - boxes[:, 1]
    valid = (ws >= BOX_MIN_SIZE) & (hs >= BOX_MIN_SIZE)
    top_s = jnp.where(valid, top_s, -jnp.inf)
    order = jnp.argsort(-top_s)
    boxes = boxes[order]
    top_s = top_s[order]

    bpad = jnp.pad(boxes, ((0, _NMS_PAD - PRENMS_TOPK), (0, 0)))
    cols = [bpad[:, i].reshape(_NMS_NB, _NMS_B, 1) for i in range(4)]
    rows = [bpad[:, i].reshape(_NMS_NB, 1, _NMS_B) for i in range(4)]
    keep = pl.pallas_call(
        _nms_body,
        out_shape=jax.ShapeDtypeStruct((_NMS_NB, _NMS_B, 1), jnp.float32),
    )(*cols, *rows)
    keepf = keep.reshape(-1)[:PRENMS_TOPK] > 0.5

    final = jnp.where(keepf & jnp.isfinite(top_s), top_s, -jnp.inf)
    out_s, out_i = lax.top_k(final, POSTNMS_TOPK)
    return boxes[out_i], out_s


# speedup vs baseline: 1.0956x; 1.0956x over previous
import numpy as np

import jax
import jax.numpy as jnp
from jax import lax
from jax.experimental import pallas as pl
from jax.experimental.pallas import tpu as pltpu

IMG_H = 800.0
IMG_W = 800.0
FEAT_H = 50
FEAT_W = 50
C = 256
NA = 9
PRENMS_TOPK = 6000
POSTNMS_TOPK = 300
NMS_IOU = 0.7
BOX_MIN_SIZE = 16.0

_PW = 52
_FLAT = _PW * _PW
_XBUF_W = 2944
_XOFF = 64
_YW = 2816

_CLIP = np.float32(np.log(1000.0 / 16.0))

_SCALES = (128.0, 256.0, 512.0)
_RATIOS = (0.5, 1.0, 2.0)
_BASE = []
for _s in _SCALES:
    for _r in _RATIOS:
        _h = _s * np.sqrt(_r)
        _w = _s / np.sqrt(_r)
        _BASE.append((np.float32(-_w / 2.0), np.float32(-_h / 2.0),
                      np.float32(_w / 2.0), np.float32(_h / 2.0)))

_NMS_B = 512
_NMS_NB = 12
_NMS_PAD = _NMS_NB * _NMS_B


def _head_body(xbuf_ref, wim_ref, wrc_ref, brpn_ref, brc_ref, out_ref,
               xim_ref):
    for ky in range(3):
        for kx in range(3):
            off = _XOFF + (ky - 1) * _PW + (kx - 1)
            k = ky * 3 + kx
            xim_ref[k * C:(k + 1) * C, :] = xbuf_ref[:, off:off + _YW]
    acc = lax.dot_general(wim_ref[:], xim_ref[:], (((1,), (0,)), ((), ())),
                          preferred_element_type=jnp.float32)
    h = jnp.maximum(acc + brpn_ref[:], 0.0)
    rc = lax.dot_general(wrc_ref[:], h, (((1,), (0,)), ((), ())),
                         preferred_element_type=jnp.float32) + brc_ref[:]

    jj = lax.broadcasted_iota(jnp.int32, (1, _YW), 1)
    gx = (16 * (jj % _PW - 1)).astype(jnp.float32)
    gy = (16 * (jj // _PW - 1)).astype(jnp.float32)

    for a in range(NA):
        bx1, by1, bx2, by2 = _BASE[a]
        x1a = gx + bx1
        y1a = gy + by1
        x2a = gx + bx2
        y2a = gy + by2
        wa = x2a - x1a
        ha = y2a - y1a
        cxa = x1a + 0.5 * wa
        cya = y1a + 0.5 * ha
        dx = rc[4 * a + 0:4 * a + 1, :]
        dy = rc[4 * a + 1:4 * a + 2, :]
        dw = jnp.minimum(rc[4 * a + 2:4 * a + 3, :], _CLIP)
        dh = jnp.minimum(rc[4 * a + 3:4 * a + 4, :], _CLIP)
        cx = cxa + dx * wa
        cy = cya + dy * ha
        w = wa * jnp.exp(dw)
        h2 = ha * jnp.exp(dh)
        x1 = jnp.minimum(jnp.maximum(cx - 0.5 * w, 0.0), IMG_W)
        y1 = jnp.minimum(jnp.maximum(cy - 0.5 * h2, 0.0), IMG_H)
        x2 = jnp.minimum(jnp.maximum(cx + 0.5 * w, 0.0), IMG_W)
        y2 = jnp.minimum(jnp.maximum(cy + 0.5 * h2, 0.0), IMG_H)
        logit = rc[4 * NA + a:4 * NA + a + 1, :]
        score = 1.0 / (1.0 + jnp.exp(-logit))
        out_ref[a:a + 1, :] = score
        out_ref[NA + a:NA + a + 1, :] = x1
        out_ref[2 * NA + a:2 * NA + a + 1, :] = y1
        out_ref[3 * NA + a:3 * NA + a + 1, :] = x2
        out_ref[4 * NA + a:4 * NA + a + 1, :] = y2


def _nms_body(x1c, y1c, x2c, y2c, x1r, y1r, x2r, y2r, keep_ref):
    keep_ref[:, :, :] = jnp.ones((_NMS_NB, _NMS_B, 1), jnp.float32)
    ii = lax.broadcasted_iota(jnp.int32, (_NMS_B, _NMS_B), 0)
    jj = lax.broadcasted_iota(jnp.int32, (_NMS_B, _NMS_B), 1)
    lt = (ii < jj).astype(jnp.float32)

    def matvec(m, k):
        return lax.dot_general(m, k, (((0,), (0,)), ((), ())),
                               preferred_element_type=jnp.float32)

    def outer(bi, _):
        ax1 = x1c[bi]
        ay1 = y1c[bi]
        ax2 = x2c[bi]
        ay2 = y2c[bi]
        area_a = jnp.maximum(ax2 - ax1, 0.0) * jnp.maximum(ay2 - ay1, 0.0)

        def iou_mask(bj):
            bx1 = x1r[bj]
            by1 = y1r[bj]
            bx2 = x2r[bj]
            by2 = y2r[bj]
            area_b = jnp.maximum(bx2 - bx1, 0.0) * jnp.maximum(by2 - by1, 0.0)
            iw = jnp.maximum(jnp.minimum(ax2, bx2) - jnp.maximum(ax1, bx1), 0.0)
            ih = jnp.maximum(jnp.minimum(ay2, by2) - jnp.maximum(ay1, by1), 0.0)
            inter = iw * ih
            iou = inter / (area_a + area_b - inter + 1e-9)
            return (iou > NMS_IOU).astype(jnp.float32)

        m_strict = iou_mask(bi) * lt
        k0 = keep_ref[bi]

        def fix_cond(c):
            kp, k = c
            return jnp.any(kp != k)

        def fix_body(c):
            _, k = c
            kn = jnp.where(matvec(m_strict, k) > 0.5, 0.0, k0)
            return (k, kn)

        first = jnp.where(matvec(m_strict, k0) > 0.5, 0.0, k0)
        _, k_fin = lax.while_loop(fix_cond, fix_body, (k0, first))
        keep_ref[bi] = k_fin

        def cross(bj, _c):
            supp = matvec(iou_mask(bj), k_fin)
            keep_ref[bj] = jnp.where(supp > 0.5, 0.0, keep_ref[bj])
            return 0

        lax.fori_loop(bi + 1, _NMS_NB, cross, 0)
        return 0

    lax.fori_loop(0, _NMS_NB, outer, 0)


def kernel(image, feat_map, target_bboxes, W_rpn, b_rpn, W_reg, b_reg,
           W_cls, b_cls):
    x = feat_map[0]
    xpad = jnp.pad(x, ((0, 0), (1, 1), (1, 1))).reshape(C, _FLAT)
    xbuf = jnp.pad(xpad, ((0, 0), (_XOFF, _XBUF_W - _FLAT - _XOFF)))
    wim = W_rpn.transpose(0, 2, 3, 1).reshape(C, 9 * C)
    wrc = jnp.concatenate([W_reg[:, :, 0, 0], W_cls[:, :, 0, 0]], axis=0)
    wrc = jnp.pad(wrc, ((0, 3), (0, 0)))
    brc = jnp.pad(jnp.concatenate([b_reg, b_cls]), (0, 3)).reshape(48, 1)
    brpn = b_rpn.reshape(C, 1)

    head = pl.pallas_call(
        _head_body,
        out_shape=jax.ShapeDtypeStruct((5 * NA, _YW), jnp.float32),
        scratch_shapes=[pltpu.VMEM((9 * C, _YW), jnp.float32)],
    )(xbuf, wim, wrc, brpn, brc)

    interior = head[:, :_FLAT].reshape(5 * NA, _PW, _PW)[:, 1:51, 1:51]

    def flat(q):
        return interior[q * NA:(q + 1) * NA].transpose(1, 2, 0).reshape(-1)

    scores = flat(0)
    boxes_all = jnp.stack([flat(1), flat(2), flat(3), flat(4)], axis=1)

    top_s, top_i = lax.top_k(scores, PRENMS_TOPK)
    boxes = boxes_all[top_i]
    ws = boxes[:, 2] - boxes[:, 0]
    hs = boxes[:, 3] - boxes[:, 1]
    valid = (ws >= BOX_MIN_SIZE) & (hs >= BOX_MIN_SIZE)
    top_s = jnp.where(valid, top_s, -jnp.inf)
    order = jnp.argsort(-top_s)
    boxes = boxes[order]
    top_s = top_s[order]

    bpad = jnp.pad(boxes, ((0, _NMS_PAD - PRENMS_TOPK), (0, 0)))
    cols = [bpad[:, i].reshape(_NMS_NB, _NMS_B, 1) for i in range(4)]
    rows = [bpad[:, i].reshape(_NMS_NB, 1, _NMS_B) for i in range(4)]
    keep = pl.pallas_call(
        _nms_body,
        out_shape=jax.ShapeDtypeStruct((_NMS_NB, _NMS_B, 1), jnp.float32),
    )(*cols, *rows)
    keepf = keep.reshape(-1)[:PRENMS_TOPK] > 0.5

    final = jnp.where(keepf & jnp.isfinite(top_s), top_s, -jnp.inf)
    out_s, out_i = lax.top_k(final, POSTNMS_TOPK)
    return boxes[out_i], out_s
